# Initial kernel scaffold; baseline (speedup 1.0000x reference)
import jax
import jax.numpy as jnp
from jax.experimental import pallas as pl


def kernel(c, r, params):
    raise NotImplementedError("write your pallas kernel here")


# speedup vs baseline: 2.0257x; 2.0257x over previous
import functools
import math

import jax
import jax.numpy as jnp
from jax.experimental import pallas as pl
from jax.experimental.pallas import tpu as pltpu

B, S = 32, 512
NUM_C = 1000
E = 320
NH = 2
ID = 416
DHM = ID // NH
HP = 256
P = NH * HP
BS = 4
K = 4
DHS = E // NH
UP = 416

_F32 = jnp.float32
_HI = jax.lax.Precision.HIGHEST
_NEG = -1e30


def _mlstm_kernel(x_ref, wln_ref, wum_ref, bum_ref, wuz_ref, buz_ref,
                  cw_ref, cb_ref, wq_ref, bq_ref, wk_ref, bk_ref,
                  wv_ref, bv_ref, wig_ref, wfg_ref, gib_ref,
                  onorm_ref, skip_ref, wd_ref, bd_ref,
                  obse_ref, osbe_ref):
    x = x_ref[0]
    mu = jnp.mean(x, -1, keepdims=True)
    xc0 = x - mu
    var = jnp.mean(xc0 * xc0, -1, keepdims=True)
    xn = xc0 * jax.lax.rsqrt(var + 1e-5) * wln_ref[...]

    xm = jnp.dot(xn, wum_ref[...], preferred_element_type=_F32) + bum_ref[...]
    z = jnp.dot(xn, wuz_ref[...], preferred_element_type=_F32) + buz_ref[...]

    cw = cw_ref[...]
    z1 = jnp.zeros((1, P), _F32)
    z2 = jnp.zeros((2, P), _F32)
    z3 = jnp.zeros((3, P), _F32)
    s1 = jnp.concatenate([z1, xm[:-1]], 0)
    s2 = jnp.concatenate([z2, xm[:-2]], 0)
    s3 = jnp.concatenate([z3, xm[:-3]], 0)
    conv = (xm * cw[3:4] + s1 * cw[2:3] + s2 * cw[1:2] + s3 * cw[0:1]
            + cb_ref[...])
    xc = conv * jax.nn.sigmoid(conv)

    q = jnp.dot(xc, wq_ref[...], preferred_element_type=_F32) + bq_ref[...]
    k = jnp.dot(xc, wk_ref[...], preferred_element_type=_F32) + bk_ref[...]
    v = jnp.dot(xm, wv_ref[...], preferred_element_type=_F32) + bv_ref[...]

    ri = jax.lax.broadcasted_iota(jnp.int32, (S, S), 0)
    cj = jax.lax.broadcasted_iota(jnp.int32, (S, S), 1)
    tril = cj <= ri
    trif = jnp.where(tril, 1.0, 0.0).astype(_F32)
    ident = jnp.where(cj == ri, 1.0, 0.0).astype(_F32)
    scale = _F32(1.0 / math.sqrt(DHM))

    wig = wig_ref[...]
    wfg = wfg_ref[...]

    heads = []
    for h in range(NH):
        igc = (jnp.sum(q * wig[3 * h:3 * h + 1] + k * wig[3 * h + 1:3 * h + 2]
                       + v * wig[3 * h + 2:3 * h + 3], -1, keepdims=True)
               + gib_ref[h])
        fgc = (jnp.sum(q * wfg[3 * h:3 * h + 1] + k * wfg[3 * h + 1:3 * h + 2]
                       + v * wfg[3 * h + 2:3 * h + 3], -1, keepdims=True)
               + gib_ref[NH + h])
        lf = jax.nn.log_sigmoid(fgc)
        F = jnp.dot(trif, lf, preferred_element_type=_F32, precision=_HI)
        Frow = jax.lax.dot_general(F, ident, (((0,), (0,)), ((), ())),
                                   preferred_element_type=_F32, precision=_HI)
        igrow = jax.lax.dot_general(igc, ident, (((0,), (0,)), ((), ())),
                                    preferred_element_type=_F32, precision=_HI)
        logD = jnp.where(tril, F - Frow + igrow, _NEG)
        maxd = jnp.max(logD, -1, keepdims=True)
        D = jnp.exp(logD - maxd)
        qh = q[:, HP * h:HP * (h + 1)]
        kh = k[:, HP * h:HP * (h + 1)]
        vh = v[:, HP * h:HP * (h + 1)]
        qk = jax.lax.dot_general(qh, kh, (((1,), (1,)), ((), ())),
                                 preferred_element_type=_F32) * scale
        C = qk * D
        rs = jnp.sum(C, -1, keepdims=True)
        norm = jnp.maximum(jnp.abs(rs), jnp.exp(-maxd)) + 1e-6
        hh = jnp.dot(C / norm, vh, preferred_element_type=_F32)
        smu = jnp.sum(hh, -1, keepdims=True) * _F32(1.0 / DHM)
        sq = jnp.sum(hh * hh, -1, keepdims=True) * _F32(1.0 / DHM)
        hv = sq - smu * smu
        hn = ((hh - smu) * jax.lax.rsqrt(hv + 1e-5)
              * onorm_ref[h:h + 1])
        heads.append(hn)

    hfull = jnp.concatenate(heads, -1)
    ht = (hfull + skip_ref[...] * xc) * (z * jax.nn.sigmoid(z))
    out = x + jnp.dot(ht, wd_ref[...], preferred_element_type=_F32) \
        + bd_ref[...]
    obse_ref[0] = out
    osbe_ref[...] = out.reshape(S, 1, 1, E)


def _mlstm_call(x, w):
    full = lambda shape: pl.BlockSpec(shape, lambda b: (0,) * len(shape))
    in_specs = [pl.BlockSpec((1, S, E), lambda b: (b, 0, 0)),
                full((1, E)), full((E, P)), full((1, P)), full((E, P)),
                full((1, P)), full((K, P)), full((1, P)),
                full((P, P)), full((1, P)), full((P, P)), full((1, P)),
                full((P, P)), full((1, P)), full((6, P)), full((6, P)),
                pl.BlockSpec(memory_space=pltpu.SMEM),
                full((NH, HP)), full((1, P)), full((P, E)), full((1, E))]
    return pl.pallas_call(
        _mlstm_kernel,
        grid=(B,),
        in_specs=in_specs,
        out_specs=[pl.BlockSpec((1, S, E), lambda b: (b, 0, 0)),
                   pl.BlockSpec((S, 1, 1, E), lambda b: (0, b, 0, 0))],
        out_shape=[jax.ShapeDtypeStruct((B, S, E), _F32),
                   jax.ShapeDtypeStruct((S, B, 1, E), _F32)],
        compiler_params=pltpu.CompilerParams(
            dimension_semantics=("parallel",),
            vmem_limit_bytes=100 * 1024 * 1024),
    )(x, w["mln"], w["wum"], w["bum"], w["wuz"], w["buz"], w["mcw"],
      w["mcb"], w["wq"], w["bq"], w["wk"], w["bk"], w["wv"], w["bv"],
      w["wig"], w["wfg"], w["gib"], w["onorm"], w["skip"], w["wd"],
      w["bd"])


_BH = B // 2
_CH = 64


def _slstm_kernel(x_hbm, wln_ref, cw_ref, cb_ref,
                  wi_ref, wf_ref, wz_ref, wo_ref,
                  ri_ref, rf_ref, rz_ref, ro_ref,
                  bg_ref, gn_ref, o_hbm,
                  x_ref, ys_ref, pi_ref, pf_ref, pz_ref, po_ref,
                  sem_in, sem_out):
    i = pl.program_id(0)
    cp_in = pltpu.make_async_copy(
        x_hbm.at[:, pl.ds(i * _BH, _BH), :], x_ref, sem_in)
    cp_in.start()
    cp_in.wait()
    cw = cw_ref[...]
    out_copies = []

    def _ln(t):
        mu = jnp.mean(t, -1, keepdims=True)
        d = t - mu
        var = jnp.mean(d * d, -1, keepdims=True)
        return d * jax.lax.rsqrt(var + 1e-5) * wln_ref[...]

    carry = (jnp.zeros((_BH, E), _F32), jnp.zeros((_BH, E), _F32),
             jnp.zeros((_BH, E), _F32), jnp.zeros((_BH, E), _F32))

    for c in range(S // _CH):
        base = c * _CH
        if c == 0:
            xe = jnp.concatenate(
                [jnp.zeros((3, _BH, E), _F32), _ln(x_ref[0:_CH])], 0)
        else:
            xe = _ln(x_ref[base - 3:base + _CH])
        xn = xe[3:]
        conv = (xn * cw[3:4] + xe[2:_CH + 2] * cw[2:3]
                + xe[1:_CH + 1] * cw[1:2] + xe[0:_CH] * cw[0:1]
                + cb_ref[...])
        xcs = conv * jax.nn.sigmoid(conv)
        x2 = xn.reshape(_CH * _BH, E)
        xc2 = xcs.reshape(_CH * _BH, E)
        pi_ref[...] = (jnp.dot(xc2, wi_ref[...], preferred_element_type=_F32)
                       + bg_ref[0:1]).reshape(_CH, _BH, E)
        pf_ref[...] = (jnp.dot(xc2, wf_ref[...], preferred_element_type=_F32)
                       + bg_ref[1:2]).reshape(_CH, _BH, E)
        pz_ref[...] = (jnp.dot(x2, wz_ref[...], preferred_element_type=_F32)
                       + bg_ref[2:3]).reshape(_CH, _BH, E)
        po_ref[...] = (jnp.dot(x2, wo_ref[...], preferred_element_type=_F32)
                       + bg_ref[3:4]).reshape(_CH, _BH, E)

        def step(t, cr):
            cst, nst, mst, yst = cr
            gi = pi_ref[pl.ds(t, 1)][0]
            gf = pf_ref[pl.ds(t, 1)][0]
            gz = pz_ref[pl.ds(t, 1)][0]
            go = po_ref[pl.ds(t, 1)][0]
            iraw = gi + jnp.dot(yst, ri_ref[...], preferred_element_type=_F32)
            fraw = gf + jnp.dot(yst, rf_ref[...], preferred_element_type=_F32)
            zraw = gz + jnp.dot(yst, rz_ref[...], preferred_element_type=_F32)
            oraw = go + jnp.dot(yst, ro_ref[...], preferred_element_type=_F32)
            lfm = mst + jax.nn.log_sigmoid(fraw)
            mnew = jnp.maximum(iraw, lfm)
            ii = jnp.exp(iraw - mnew)
            ff = jnp.exp(lfm - mnew)
            cnew = ff * cst + ii * jnp.tanh(zraw)
            nnew = ff * nst + ii
            ynew = jax.nn.sigmoid(oraw) * cnew / nnew
            ys_ref[pl.ds(base + t, 1)] = ynew.reshape(1, _BH, E)
            return (cnew, nnew, mnew, ynew)

        carry = jax.lax.fori_loop(0, _CH, step, carry)

        y = ys_ref[base:base + _CH].reshape(_CH * _BH, E)
        lmask = jax.lax.broadcasted_iota(jnp.int32, (1, E), 1) < DHS
        y2 = y * y
        s_all = jnp.sum(y, -1, keepdims=True)
        q_all = jnp.sum(y2, -1, keepdims=True)
        s0 = jnp.sum(jnp.where(lmask, y, 0.0), -1, keepdims=True)
        q0 = jnp.sum(jnp.where(lmask, y2, 0.0), -1, keepdims=True)
        mu = jnp.where(lmask, s0, s_all - s0) * _F32(1.0 / DHS)
        ex2 = jnp.where(lmask, q0, q_all - q0) * _F32(1.0 / DHS)
        var = ex2 - mu * mu
        yn = (y - mu) * jax.lax.rsqrt(var + 1e-5) * gn_ref[...]
        ys_ref[base:base + _CH] = yn.reshape(_CH, _BH, E)
        cp_out = pltpu.make_async_copy(
            ys_ref.at[base:base + _CH],
            o_hbm.at[base:base + _CH, pl.ds(i * _BH, _BH), :], sem_out)
        cp_out.start()
        out_copies.append(cp_out)

    for cp_out in out_copies:
        cp_out.wait()


def _slstm_call(x_sbe, w):
    full = lambda shape: pl.BlockSpec(shape, lambda b: (0,) * len(shape))
    in_specs = [pl.BlockSpec(memory_space=pl.ANY),
                full((1, E)), full((K, E)), full((1, E)),
                full((E, E)), full((E, E)), full((E, E)), full((E, E)),
                full((E, E)), full((E, E)), full((E, E)), full((E, E)),
                full((4, E)), full((1, E))]
    return pl.pallas_call(
        _slstm_kernel,
        grid=(2,),
        in_specs=in_specs,
        out_specs=pl.BlockSpec(memory_space=pl.ANY),
        out_shape=jax.ShapeDtypeStruct((S, B, E), _F32),
        scratch_shapes=[pltpu.VMEM((S, _BH, E), _F32),
                        pltpu.VMEM((S, _BH, E), _F32)]
        + [pltpu.VMEM((_CH, _BH, E), _F32)] * 4
        + [pltpu.SemaphoreType.DMA, pltpu.SemaphoreType.DMA],
        compiler_params=pltpu.CompilerParams(
            dimension_semantics=("parallel",),
            vmem_limit_bytes=100 * 1024 * 1024),
    )(x_sbe, w["sln"], w["scw"], w["scb"], w["swi"], w["swf"], w["swz"],
      w["swo"], w["sri"], w["srf"], w["srz"], w["sro"], w["sbg"], w["sgn"])


def _head_kernel(x1_ref, y_ref, pid_ref, rm_ref, qe_ref,
                 fln_ref, wfg_ref, bfg_ref, wfu_ref, bfu_ref,
                 wfd_ref, bfd_ref, pln_ref,
                 w1a_ref, w1b_ref, w1c_ref, w1d_ref, b1_ref,
                 w2_ref, b2_ref, w3_ref, b3_ref, o_ref):
    x2 = x1_ref[0] + y_ref[:, 0, 0, :]

    def _ln(t, wref):
        mu = jnp.mean(t, -1, keepdims=True)
        d = t - mu
        var = jnp.mean(d * d, -1, keepdims=True)
        return d * jax.lax.rsqrt(var + 1e-5) * wref[...]

    t = _ln(x2, fln_ref)
    g = jnp.dot(t, wfg_ref[...], preferred_element_type=_F32) + bfg_ref[...]
    u = jnp.dot(t, wfu_ref[...], preferred_element_type=_F32) + bfu_ref[...]
    h = jnp.dot(jnp.maximum(g, 0.0) * u, wfd_ref[...],
                preferred_element_type=_F32) + bfd_ref[...]
    x3 = x2 + h
    d = _ln(x3, pln_ref)
    d = jnp.where(jnp.isnan(d), 0.0, d)
    d = jnp.where(d == jnp.inf, 1.0, d)
    d = jnp.where(d == -jnp.inf, -1.0, d)

    pid = pid_ref[0]
    rm = rm_ref[0]
    qe = qe_ref[0]
    dm = d - pid
    fam = d * rm
    unf = d * (1.0 - rm)
    h1 = (jnp.dot(dm, w1a_ref[...], preferred_element_type=_F32)
          + jnp.dot(qe, w1b_ref[...], preferred_element_type=_F32)
          + jnp.dot(fam, w1c_ref[...], preferred_element_type=_F32)
          + jnp.dot(unf, w1d_ref[...], preferred_element_type=_F32)
          + b1_ref[...])
    h1 = jnp.maximum(h1, 0.0)
    h2 = jnp.maximum(jnp.dot(h1, w2_ref[...], preferred_element_type=_F32)
                     + b2_ref[...], 0.0)
    o_ref[0] = jax.nn.sigmoid(
        jnp.dot(h2, w3_ref[...], preferred_element_type=_F32) + b3_ref[...])


def _head_call(x1, y_sbe, pid, rm, qe, w):
    full = lambda shape: pl.BlockSpec(shape, lambda b: (0,) * len(shape))
    in_specs = [pl.BlockSpec((1, S, E), lambda b: (b, 0, 0)),
                pl.BlockSpec((S, 1, 1, E), lambda b: (0, b, 0, 0)),
                pl.BlockSpec((1, S, 1), lambda b: (b, 0, 0)),
                pl.BlockSpec((1, S, 1), lambda b: (b, 0, 0)),
                pl.BlockSpec((1, S, E), lambda b: (b, 0, 0)),
                full((1, E)), full((E, UP)), full((1, UP)), full((E, UP)),
                full((1, UP)), full((UP, E)), full((1, E)), full((1, E)),
                full((E, 2 * E)), full((E, 2 * E)), full((E, 2 * E)),
                full((E, 2 * E)), full((1, 2 * E)),
                full((2 * E, E)), full((1, E)),
                full((E, NUM_C)), full((1, NUM_C))]
    return pl.pallas_call(
        _head_kernel,
        grid=(B,),
        in_specs=in_specs,
        out_specs=pl.BlockSpec((1, S, NUM_C), lambda b: (b, 0, 0)),
        out_shape=jax.ShapeDtypeStruct((B, S, NUM_C), _F32),
        compiler_params=pltpu.CompilerParams(
            dimension_semantics=("parallel",),
            vmem_limit_bytes=100 * 1024 * 1024),
    )(x1, y_sbe, pid, rm, qe, w["fln"], w["wffg"], w["bffg"], w["wffu"],
      w["bffu"], w["wffd"], w["bffd"], w["pln"], w["w1a"], w["w1b"],
      w["w1c"], w["w1d"], w["b1"], w["w2"], w["b2"], w["w3"], w["b3"])


def _pad_cols(w):
    z = jnp.zeros(w.shape[:-1] + (P,), _F32)
    z = z.at[..., :DHM].set(w[..., :DHM])
    return z.at[..., HP:HP + DHM].set(w[..., DHM:])


def _pad_rows(w):
    z = jnp.zeros((P,) + w.shape[1:], _F32)
    z = z.at[:DHM].set(w[:DHM])
    return z.at[HP:HP + DHM].set(w[DHM:])


def _dense_headwise(w):
    nb = w.shape[0]
    eye = jnp.eye(nb, dtype=_F32)
    return jnp.einsum('boi,bc->bico', w, eye).reshape(nb * BS, nb * BS)


def _build_weights(p):
    m, s = p["m"], p["s"]
    wup = m["proj_up"]["w"]
    bup = m["proj_up"]["b"]
    wig = m["ig"]["w"]
    wfg = m["fg"]["w"]
    wig6 = jnp.stack([_pad_cols(wig[i * ID:(i + 1) * ID, h])
                      for h in range(NH) for i in range(3)])
    wfg6 = jnp.stack([_pad_cols(wfg[i * ID:(i + 1) * ID, h])
                      for h in range(NH) for i in range(3)])
    gib = jnp.concatenate([m["ig"]["b"], m["fg"]["b"]])
    wg4 = s["Wg"]
    rg4 = s["Rg"]

    def sdense(w2):
        z = jnp.zeros((E, E), _F32)
        for h in range(NH):
            z = z.at[h * DHS:(h + 1) * DHS, h * DHS:(h + 1) * DHS].set(w2[h])
        return z

    w = {
        "mln": m["ln"].reshape(1, E),
        "wum": _pad_cols(wup[:, :ID]),
        "bum": _pad_cols(bup[:ID]).reshape(1, P),
        "wuz": _pad_cols(wup[:, ID:]),
        "buz": _pad_cols(bup[ID:]).reshape(1, P),
        "mcw": _pad_cols(m["conv_w"][:, 0, :]),
        "mcb": _pad_cols(m["conv_b"]).reshape(1, P),
        "wq": _pad_rows(_pad_cols(_dense_headwise(m["q_w"]))),
        "bq": _pad_cols(m["q_b"]).reshape(1, P),
        "wk": _pad_rows(_pad_cols(_dense_headwise(m["k_w"]))),
        "bk": _pad_cols(m["k_b"]).reshape(1, P),
        "wv": _pad_rows(_pad_cols(_dense_headwise(m["v_w"]))),
        "bv": _pad_cols(m["v_b"]).reshape(1, P),
        "wig": wig6, "wfg": wfg6, "gib": gib,
        "onorm": jnp.zeros((NH, HP), _F32).at[:, :DHM].set(
            m["outnorm"].reshape(NH, DHM)),
        "skip": _pad_cols(m["skip"]).reshape(1, P),
        "wd": _pad_rows(m["proj_down"]["w"]),
        "bd": m["proj_down"]["b"].reshape(1, E),
        "sln": s["ln"].reshape(1, E),
        "scw": s["conv_w"][:, 0, :],
        "scb": s["conv_b"].reshape(1, E),
        "swi": sdense(wg4[0]), "swf": sdense(wg4[1]),
        "swz": sdense(wg4[2]), "swo": sdense(wg4[3]),
        "sri": sdense(rg4[0]), "srf": sdense(rg4[1]),
        "srz": sdense(rg4[2]), "sro": sdense(rg4[3]),
        "sbg": s["bg"].reshape(4, E),
        "sgn": s["gn"].reshape(1, E),
        "fln": s["ffn_ln"].reshape(1, E),
        "wffg": s["ffn_up"]["w"][:, :UP],
        "bffg": s["ffn_up"]["b"][:UP].reshape(1, UP),
        "wffu": s["ffn_up"]["w"][:, UP:],
        "bffu": s["ffn_up"]["b"][UP:].reshape(1, UP),
        "wffd": s["ffn_down"]["w"],
        "bffd": s["ffn_down"]["b"].reshape(1, E),
        "pln": p["post_norm"].reshape(1, E),
        "w1a": p["out1"]["w"][:E], "w1b": p["out1"]["w"][E:2 * E],
        "w1c": p["out1"]["w"][2 * E:3 * E], "w1d": p["out1"]["w"][3 * E:],
        "b1": p["out1"]["b"].reshape(1, 2 * E),
        "w2": p["out2"]["w"], "b2": p["out2"]["b"].reshape(1, E),
        "w3": p["out3"]["w"], "b3": p["out3"]["b"].reshape(1, NUM_C),
    }
    return w


@jax.jit
def _run(c, r, params):
    p = params
    cc = jnp.clip(c, 0, NUM_C - 1)
    rr = jnp.clip(r, 0, 1)
    qe = p["q_embed"][cc]
    pid = p["difficult"][cc]
    qa = p["qa_embed"][rr] + qe + pid * p["qa_embed_diff"][rr + 2 * cc]
    q_emb = qe + pid * p["q_embed_diff"][cc]
    rm = rr.astype(_F32)[..., None]

    w = _build_weights(p)
    x1_bse, x1_sbe4 = _mlstm_call(qa, w)
    ys_sbe = _slstm_call(x1_sbe4.reshape(S, B, E), w)
    return _head_call(x1_bse, ys_sbe.reshape(S, B, 1, E), pid, rm, q_emb, w)


def kernel(c, r, params):
    return _run(c, r, params)
